# Initial kernel scaffold; baseline (speedup 1.0000x reference)
import jax
import jax.numpy as jnp
from jax.experimental import pallas as pl


def kernel(x, edge_index, edge_attr, batch_idx, W_emb1, b_emb1, W_emb2, b_emb2, W_gcn0, b_gcn0, W_gcn1, b_gcn1, W_gcn2, b_gcn2, W_out, b_out):
    raise NotImplementedError("write your pallas kernel here")


# speedup vs baseline: 8.6266x; 8.6266x over previous
import functools

import jax
import jax.numpy as jnp
from jax import lax
from jax.experimental import pallas as pl
from jax.experimental.pallas import tpu as pltpu
from jax.experimental.pallas import tpu_sc as plsc

N_NODES = 10000
N_EDGES = 320000
HID = 128
NGRAPH = 64

_INFO = plsc.get_sparse_core_info()
NC = _INFO.num_cores
NS = _INFO.num_subcores
NW = NC * NS
EPW = N_EDGES // NW
CH = 80
NCHUNK = EPW // CH
RPT = 624
TAIL = N_NODES - RPT * NS
TAIL_OFF = RPT * NS


def _sc_agg_body(g_hbm, src_hbm, dst_hbm, out_hbm, src_v, dst_v, rows_v,
                 acc_sh, sem):
    c = lax.axis_index("c")
    s = lax.axis_index("s")
    wid = c * NS + s
    ebase = wid * EPW
    pltpu.sync_copy(g_hbm.at[pl.ds(s * RPT, RPT)],
                    acc_sh.at[pl.ds(s * RPT, RPT)])

    @pl.when(s == 0)
    def _():
        pltpu.sync_copy(g_hbm.at[pl.ds(TAIL_OFF, TAIL)],
                        acc_sh.at[pl.ds(TAIL_OFF, TAIL)])

    plsc.subcore_barrier()

    def body(i, carry):
        off = ebase + i * CH
        pltpu.sync_copy(src_hbm.at[pl.ds(off, CH)], src_v)
        pltpu.sync_copy(dst_hbm.at[pl.ds(off, CH)], dst_v)
        pltpu.async_copy(g_hbm.at[src_v], rows_v, sem).wait()
        pltpu.sync_copy(rows_v, acc_sh.at[dst_v], add=True)
        return carry

    lax.fori_loop(0, NCHUNK, body, 0)
    plsc.subcore_barrier()
    pltpu.sync_copy(acc_sh.at[pl.ds(s * RPT, RPT)],
                    out_hbm.at[c, pl.ds(s * RPT, RPT)])

    @pl.when(s == 0)
    def _():
        pltpu.sync_copy(acc_sh.at[pl.ds(TAIL_OFF, TAIL)],
                        out_hbm.at[c, pl.ds(TAIL_OFF, TAIL)])


_sc_agg = functools.partial(
    pl.kernel,
    out_type=jax.ShapeDtypeStruct((NC, N_NODES, HID), jnp.float32),
    mesh=plsc.VectorSubcoreMesh(core_axis_name="c", subcore_axis_name="s"),
    scratch_types=[
        pltpu.VMEM((CH,), jnp.int32),
        pltpu.VMEM((CH,), jnp.int32),
        pltpu.VMEM((CH, HID), jnp.float32),
        pltpu.VMEM_SHARED((N_NODES, HID), jnp.float32),
        pltpu.SemaphoreType.DMA,
    ],
)(_sc_agg_body)


def _tc_embed_body(x_ref, accd_ref, w1_ref, b1_ref, w2_ref, b2_ref, w0_ref,
                   g_ref, dinv_ref):
    deg = accd_ref[0][:, 0:1] + accd_ref[1][:, 0:1] - 1.0
    dinv = 1.0 / jnp.sqrt(deg)
    h = jnp.maximum(
        jnp.dot(x_ref[...], w1_ref[...], preferred_element_type=jnp.float32)
        + b1_ref[...], 0.0)
    h = jnp.dot(h, w2_ref[...], preferred_element_type=jnp.float32) + b2_ref[...]
    g_ref[...] = dinv * jnp.dot(h, w0_ref[...],
                                preferred_element_type=jnp.float32)
    dinv_ref[...] = dinv


def _tc_layer_body(acc_ref, g_ref, dinv_ref, b_ref, w_ref, gout_ref):
    dinv = dinv_ref[...]
    h = acc_ref[0] + acc_ref[1] - g_ref[...]
    h = jnp.maximum(dinv * h + b_ref[...], 0.0)
    gout_ref[...] = dinv * jnp.dot(h, w_ref[...],
                                   preferred_element_type=jnp.float32)


def _tc_final_body(acc_ref, g_ref, dinv_ref, b_ref, bi_ref, wout_ref,
                   bout_ref, out_ref):
    dinv = dinv_ref[...]
    h = acc_ref[0] + acc_ref[1] - g_ref[...]
    h = jnp.maximum(dinv * h + b_ref[...], 0.0)
    onehot = (bi_ref[...] == lax.broadcasted_iota(
        jnp.int32, (N_NODES, NGRAPH), 1)).astype(jnp.float32)
    pooled = lax.dot_general(onehot, h, (((0,), (0,)), ((), ())),
                             preferred_element_type=jnp.float32)
    counts = lax.dot_general(onehot, jnp.ones((N_NODES, 1), jnp.float32),
                             (((0,), (0,)), ((), ())),
                             preferred_element_type=jnp.float32)
    pooled = pooled / jnp.maximum(counts, 1.0)
    out_ref[...] = jnp.dot(pooled, wout_ref[...],
                           preferred_element_type=jnp.float32) + bout_ref[...]


def kernel(x, edge_index, edge_attr, batch_idx, W_emb1, b_emb1, W_emb2,
           b_emb2, W_gcn0, b_gcn0, W_gcn1, b_gcn1, W_gcn2, b_gcn2, W_out,
           b_out):
    del edge_attr
    src = edge_index[0]
    dst = edge_index[1]
    b1 = b_emb1.reshape(1, HID)
    b2 = b_emb2.reshape(1, HID)
    bg0 = b_gcn0.reshape(1, HID)
    bg1 = b_gcn1.reshape(1, HID)
    bg2 = b_gcn2.reshape(1, HID)
    bo = b_out.reshape(1, 1)
    bi = batch_idx.reshape(N_NODES, 1)

    ones_tab = jnp.ones((N_NODES, HID), jnp.float32)
    acc_deg = _sc_agg(ones_tab, src, dst)

    g0, dinv = pl.pallas_call(
        _tc_embed_body,
        out_shape=(jax.ShapeDtypeStruct((N_NODES, HID), jnp.float32),
                   jax.ShapeDtypeStruct((N_NODES, 1), jnp.float32)),
    )(x, acc_deg, W_emb1, b1, W_emb2, b2, W_gcn0)

    acc0 = _sc_agg(g0, src, dst)
    g1 = pl.pallas_call(
        _tc_layer_body,
        out_shape=jax.ShapeDtypeStruct((N_NODES, HID), jnp.float32),
    )(acc0, g0, dinv, bg0, W_gcn1)

    acc1 = _sc_agg(g1, src, dst)
    g2 = pl.pallas_call(
        _tc_layer_body,
        out_shape=jax.ShapeDtypeStruct((N_NODES, HID), jnp.float32),
    )(acc1, g1, dinv, bg1, W_gcn2)

    acc2 = _sc_agg(g2, src, dst)
    out = pl.pallas_call(
        _tc_final_body,
        out_shape=jax.ShapeDtypeStruct((NGRAPH, 1), jnp.float32),
    )(acc2, g2, dinv, bg2, bi, W_out, bo)
    return out


# speedup vs baseline: 15.7412x; 1.8247x over previous
import functools

import jax
import jax.numpy as jnp
from jax import lax
from jax.experimental import pallas as pl
from jax.experimental.pallas import tpu as pltpu
from jax.experimental.pallas import tpu_sc as plsc

N_NODES = 10000
N_EDGES = 320000
HID = 128
NGRAPH = 64

_INFO = plsc.get_sparse_core_info()
NC = _INFO.num_cores
NS = _INFO.num_subcores
NW = NC * NS
EPW = N_EDGES // NW
CH = 80
NCHUNK = EPW // CH
NBUF = 2
NQUAD = NCHUNK // NBUF
RPT = 624
TAIL = N_NODES - RPT * NS
TAIL_OFF = RPT * NS


def _sc_agg_body(g_hbm, src_hbm, dst_hbm, out_hbm, src_all, dst_all,
                 rows_a, rows_b, acc_sh, isem0, isem1, sem_a, sem_b):
    c = lax.axis_index("c")
    s = lax.axis_index("s")
    wid = c * NS + s
    rows = [rows_a, rows_b]
    sems = [sem_a, sem_b]

    ih0 = pltpu.async_copy(src_hbm.at[wid], src_all, isem0)
    ih1 = pltpu.async_copy(dst_hbm.at[wid], dst_all, isem1)

    pltpu.sync_copy(g_hbm.at[pl.ds(s * RPT, RPT)],
                    acc_sh.at[pl.ds(s * RPT, RPT)])

    @pl.when(s == 0)
    def _():
        pltpu.sync_copy(g_hbm.at[pl.ds(TAIL_OFF, TAIL)],
                        acc_sh.at[pl.ds(TAIL_OFF, TAIL)])

    ih0.wait()
    ih1.wait()
    plsc.subcore_barrier()

    def quad(k, carry):
        i0 = k * NBUF
        handles = []
        for b in range(NBUF):
            off = pl.multiple_of((i0 + b) * CH, 8)
            handles.append(
                pltpu.async_copy(g_hbm.at[src_all.at[pl.ds(off, CH)]],
                                 rows[b], sems[b]))
        for b in range(NBUF):
            handles[b].wait()
            pltpu.sync_copy(rows[b], acc_sh.at[dst_all.at[i0 + b]], add=True)
        return carry

    lax.fori_loop(0, NQUAD, quad, 0)
    for i in range(NQUAD * NBUF, NCHUNK):
        pltpu.async_copy(g_hbm.at[src_all.at[pl.ds(i * CH, CH)]], rows_a,
                         sem_a).wait()
        pltpu.sync_copy(rows_a, acc_sh.at[dst_all.at[i]], add=True)

    plsc.subcore_barrier()
    pltpu.sync_copy(acc_sh.at[pl.ds(s * RPT, RPT)],
                    out_hbm.at[c, pl.ds(s * RPT, RPT)])

    @pl.when(s == 0)
    def _():
        pltpu.sync_copy(acc_sh.at[pl.ds(TAIL_OFF, TAIL)],
                        out_hbm.at[c, pl.ds(TAIL_OFF, TAIL)])


_sc_agg = functools.partial(
    pl.kernel,
    out_type=jax.ShapeDtypeStruct((NC, N_NODES, HID), jnp.float32),
    mesh=plsc.VectorSubcoreMesh(core_axis_name="c", subcore_axis_name="s"),
    scratch_types=[
        pltpu.VMEM((EPW,), jnp.int32),
        pltpu.VMEM((NCHUNK, CH), jnp.int32),
        pltpu.VMEM((CH, HID), jnp.float32),
        pltpu.VMEM((CH, HID), jnp.float32),
        pltpu.VMEM_SHARED((N_NODES, HID), jnp.float32),
        pltpu.SemaphoreType.DMA,
        pltpu.SemaphoreType.DMA,
        pltpu.SemaphoreType.DMA,
        pltpu.SemaphoreType.DMA,
    ],
)(_sc_agg_body)


def _tc_embed_body(x_ref, accd_ref, w1_ref, b1_ref, w2_ref, b2_ref, w0_ref,
                   g_ref, dinv_ref):
    deg = accd_ref[0][:, 0:1] + accd_ref[1][:, 0:1] - 1.0
    dinv = 1.0 / jnp.sqrt(deg)
    h = jnp.maximum(
        jnp.dot(x_ref[...], w1_ref[...], preferred_element_type=jnp.float32)
        + b1_ref[...], 0.0)
    h = jnp.dot(h, w2_ref[...], preferred_element_type=jnp.float32) + b2_ref[...]
    g_ref[...] = dinv * jnp.dot(h, w0_ref[...],
                                preferred_element_type=jnp.float32)
    dinv_ref[...] = dinv


def _tc_layer_body(acc_ref, g_ref, dinv_ref, b_ref, w_ref, gout_ref):
    dinv = dinv_ref[...]
    h = acc_ref[0] + acc_ref[1] - g_ref[...]
    h = jnp.maximum(dinv * h + b_ref[...], 0.0)
    gout_ref[...] = dinv * jnp.dot(h, w_ref[...],
                                   preferred_element_type=jnp.float32)


def _tc_final_body(acc_ref, g_ref, dinv_ref, b_ref, bi_ref, wout_ref,
                   bout_ref, out_ref):
    dinv = dinv_ref[...]
    h = acc_ref[0] + acc_ref[1] - g_ref[...]
    h = jnp.maximum(dinv * h + b_ref[...], 0.0)
    onehot = (bi_ref[...] == lax.broadcasted_iota(
        jnp.int32, (N_NODES, NGRAPH), 1)).astype(jnp.float32)
    pooled = lax.dot_general(onehot, h, (((0,), (0,)), ((), ())),
                             preferred_element_type=jnp.float32)
    counts = lax.dot_general(onehot, jnp.ones((N_NODES, 1), jnp.float32),
                             (((0,), (0,)), ((), ())),
                             preferred_element_type=jnp.float32)
    pooled = pooled / jnp.maximum(counts, 1.0)
    out_ref[...] = jnp.dot(pooled, wout_ref[...],
                           preferred_element_type=jnp.float32) + bout_ref[...]


def kernel(x, edge_index, edge_attr, batch_idx, W_emb1, b_emb1, W_emb2,
           b_emb2, W_gcn0, b_gcn0, W_gcn1, b_gcn1, W_gcn2, b_gcn2, W_out,
           b_out):
    del edge_attr
    src = edge_index[0].reshape(NW, EPW)
    dst = edge_index[1].reshape(NW, NCHUNK, CH)
    b1 = b_emb1.reshape(1, HID)
    b2 = b_emb2.reshape(1, HID)
    bg0 = b_gcn0.reshape(1, HID)
    bg1 = b_gcn1.reshape(1, HID)
    bg2 = b_gcn2.reshape(1, HID)
    bo = b_out.reshape(1, 1)
    bi = batch_idx.reshape(N_NODES, 1)

    ones_tab = jnp.ones((N_NODES, HID), jnp.float32)
    acc_deg = _sc_agg(ones_tab, src, dst)

    g0, dinv = pl.pallas_call(
        _tc_embed_body,
        out_shape=(jax.ShapeDtypeStruct((N_NODES, HID), jnp.float32),
                   jax.ShapeDtypeStruct((N_NODES, 1), jnp.float32)),
    )(x, acc_deg, W_emb1, b1, W_emb2, b2, W_gcn0)

    acc0 = _sc_agg(g0, src, dst)
    g1 = pl.pallas_call(
        _tc_layer_body,
        out_shape=jax.ShapeDtypeStruct((N_NODES, HID), jnp.float32),
    )(acc0, g0, dinv, bg0, W_gcn1)

    acc1 = _sc_agg(g1, src, dst)
    g2 = pl.pallas_call(
        _tc_layer_body,
        out_shape=jax.ShapeDtypeStruct((N_NODES, HID), jnp.float32),
    )(acc1, g1, dinv, bg1, W_gcn2)

    acc2 = _sc_agg(g2, src, dst)
    out = pl.pallas_call(
        _tc_final_body,
        out_shape=jax.ShapeDtypeStruct((NGRAPH, 1), jnp.float32),
    )(acc2, g2, dinv, bg2, bi, W_out, bo)
    return out


# speedup vs baseline: 16.1035x; 1.0230x over previous
import functools

import jax
import jax.numpy as jnp
from jax import lax
from jax.experimental import pallas as pl
from jax.experimental.pallas import tpu as pltpu
from jax.experimental.pallas import tpu_sc as plsc

N_NODES = 10000
N_EDGES = 320000
HID = 128
NGRAPH = 64

_INFO = plsc.get_sparse_core_info()
NC = _INFO.num_cores
NS = _INFO.num_subcores
NW = NC * NS
EPW = N_EDGES // NW
CH = 80
NCHUNK = EPW // CH
NBUF = 2
NQUAD = NCHUNK // NBUF
RPT = 624
TAIL = N_NODES - RPT * NS
TAIL_OFF = RPT * NS


def _sc_agg_body(g_hbm, src_hbm, dst_hbm, out_hbm, src_all, dst_all,
                 rows_a, rows_b, acc_sh, isem0, isem1, sem_a, sem_b,
                 ssem_a, ssem_b):
    c = lax.axis_index("c")
    s = lax.axis_index("s")
    wid = c * NS + s
    rows = [rows_a, rows_b]
    sems = [sem_a, sem_b]
    ssems = [ssem_a, ssem_b]

    ih0 = pltpu.async_copy(src_hbm.at[wid], src_all, isem0)
    ih1 = pltpu.async_copy(dst_hbm.at[wid], dst_all, isem1)

    pltpu.sync_copy(g_hbm.at[pl.ds(s * RPT, RPT)],
                    acc_sh.at[pl.ds(s * RPT, RPT)])

    @pl.when(s == 0)
    def _():
        pltpu.sync_copy(g_hbm.at[pl.ds(TAIL_OFF, TAIL)],
                        acc_sh.at[pl.ds(TAIL_OFF, TAIL)])

    ih0.wait()
    ih1.wait()
    plsc.subcore_barrier()

    def quad(k, carry):
        i0 = k * NBUF
        handles = []
        for b in range(NBUF):
            off = pl.multiple_of((i0 + b) * CH, 8)
            handles.append(
                pltpu.async_copy(g_hbm.at[src_all.at[pl.ds(off, CH)]],
                                 rows[b], sems[b]))
        scat = []
        for b in range(NBUF):
            handles[b].wait()
            scat.append(
                pltpu.async_copy(rows[b], acc_sh.at[dst_all.at[i0 + b]],
                                 ssems[b], add=True))
        for b in range(NBUF):
            scat[b].wait()
        return carry

    lax.fori_loop(0, NQUAD, quad, 0)
    for i in range(NQUAD * NBUF, NCHUNK):
        pltpu.async_copy(g_hbm.at[src_all.at[pl.ds(i * CH, CH)]], rows_a,
                         sem_a).wait()
        pltpu.sync_copy(rows_a, acc_sh.at[dst_all.at[i]], add=True)

    plsc.subcore_barrier()
    pltpu.sync_copy(acc_sh.at[pl.ds(s * RPT, RPT)],
                    out_hbm.at[c, pl.ds(s * RPT, RPT)])

    @pl.when(s == 0)
    def _():
        pltpu.sync_copy(acc_sh.at[pl.ds(TAIL_OFF, TAIL)],
                        out_hbm.at[c, pl.ds(TAIL_OFF, TAIL)])


_sc_agg = functools.partial(
    pl.kernel,
    out_type=jax.ShapeDtypeStruct((NC, N_NODES, HID), jnp.float32),
    mesh=plsc.VectorSubcoreMesh(core_axis_name="c", subcore_axis_name="s"),
    scratch_types=[
        pltpu.VMEM((EPW,), jnp.int32),
        pltpu.VMEM((NCHUNK, CH), jnp.int32),
        pltpu.VMEM((CH, HID), jnp.float32),
        pltpu.VMEM((CH, HID), jnp.float32),
        pltpu.VMEM_SHARED((N_NODES, HID), jnp.float32),
        pltpu.SemaphoreType.DMA,
        pltpu.SemaphoreType.DMA,
        pltpu.SemaphoreType.DMA,
        pltpu.SemaphoreType.DMA,
        pltpu.SemaphoreType.DMA,
        pltpu.SemaphoreType.DMA,
    ],
)(_sc_agg_body)


def _tc_embed_body(x_ref, accd_ref, w1_ref, b1_ref, w2_ref, b2_ref, w0_ref,
                   g_ref, dinv_ref):
    deg = accd_ref[0][:, 0:1] + accd_ref[1][:, 0:1] - 1.0
    dinv = 1.0 / jnp.sqrt(deg)
    h = jnp.maximum(
        jnp.dot(x_ref[...], w1_ref[...], preferred_element_type=jnp.float32)
        + b1_ref[...], 0.0)
    h = jnp.dot(h, w2_ref[...], preferred_element_type=jnp.float32) + b2_ref[...]
    g_ref[...] = dinv * jnp.dot(h, w0_ref[...],
                                preferred_element_type=jnp.float32)
    dinv_ref[...] = dinv


def _tc_layer_body(acc_ref, g_ref, dinv_ref, b_ref, w_ref, gout_ref):
    dinv = dinv_ref[...]
    h = acc_ref[0] + acc_ref[1] - g_ref[...]
    h = jnp.maximum(dinv * h + b_ref[...], 0.0)
    gout_ref[...] = dinv * jnp.dot(h, w_ref[...],
                                   preferred_element_type=jnp.float32)


def _tc_final_body(acc_ref, g_ref, dinv_ref, b_ref, bi_ref, wout_ref,
                   bout_ref, out_ref):
    dinv = dinv_ref[...]
    h = acc_ref[0] + acc_ref[1] - g_ref[...]
    h = jnp.maximum(dinv * h + b_ref[...], 0.0)
    onehot = (bi_ref[...] == lax.broadcasted_iota(
        jnp.int32, (N_NODES, NGRAPH), 1)).astype(jnp.float32)
    pooled = lax.dot_general(onehot, h, (((0,), (0,)), ((), ())),
                             preferred_element_type=jnp.float32)
    counts = lax.dot_general(onehot, jnp.ones((N_NODES, 1), jnp.float32),
                             (((0,), (0,)), ((), ())),
                             preferred_element_type=jnp.float32)
    pooled = pooled / jnp.maximum(counts, 1.0)
    out_ref[...] = jnp.dot(pooled, wout_ref[...],
                           preferred_element_type=jnp.float32) + bout_ref[...]


def kernel(x, edge_index, edge_attr, batch_idx, W_emb1, b_emb1, W_emb2,
           b_emb2, W_gcn0, b_gcn0, W_gcn1, b_gcn1, W_gcn2, b_gcn2, W_out,
           b_out):
    del edge_attr
    src = edge_index[0].reshape(NW, EPW)
    dst = edge_index[1].reshape(NW, NCHUNK, CH)
    b1 = b_emb1.reshape(1, HID)
    b2 = b_emb2.reshape(1, HID)
    bg0 = b_gcn0.reshape(1, HID)
    bg1 = b_gcn1.reshape(1, HID)
    bg2 = b_gcn2.reshape(1, HID)
    bo = b_out.reshape(1, 1)
    bi = batch_idx.reshape(N_NODES, 1)

    ones_tab = jnp.ones((N_NODES, HID), jnp.float32)
    acc_deg = _sc_agg(ones_tab, src, dst)

    g0, dinv = pl.pallas_call(
        _tc_embed_body,
        out_shape=(jax.ShapeDtypeStruct((N_NODES, HID), jnp.float32),
                   jax.ShapeDtypeStruct((N_NODES, 1), jnp.float32)),
    )(x, acc_deg, W_emb1, b1, W_emb2, b2, W_gcn0)

    acc0 = _sc_agg(g0, src, dst)
    g1 = pl.pallas_call(
        _tc_layer_body,
        out_shape=jax.ShapeDtypeStruct((N_NODES, HID), jnp.float32),
    )(acc0, g0, dinv, bg0, W_gcn1)

    acc1 = _sc_agg(g1, src, dst)
    g2 = pl.pallas_call(
        _tc_layer_body,
        out_shape=jax.ShapeDtypeStruct((N_NODES, HID), jnp.float32),
    )(acc1, g1, dinv, bg1, W_gcn2)

    acc2 = _sc_agg(g2, src, dst)
    out = pl.pallas_call(
        _tc_final_body,
        out_shape=jax.ShapeDtypeStruct((NGRAPH, 1), jnp.float32),
    )(acc2, g2, dinv, bg2, bi, W_out, bo)
    return out


# speedup vs baseline: 19.7513x; 1.2265x over previous
import functools

import jax
import jax.numpy as jnp
from jax import lax
from jax.experimental import pallas as pl
from jax.experimental.pallas import tpu as pltpu
from jax.experimental.pallas import tpu_sc as plsc

N_NODES = 10000
N_EDGES = 320000
HID = 128
NGRAPH = 64

_INFO = plsc.get_sparse_core_info()
NC = _INFO.num_cores
NS = _INFO.num_subcores
NW = NC * NS
EPW = N_EDGES // NW
CH = 80
NCHUNK = EPW // CH
NBUF = 2
NQUAD = NCHUNK // NBUF
RPT = 624
TAIL = N_NODES - RPT * NS
TAIL_OFF = RPT * NS


def _sc_agg_body(g_hbm, src_hbm, dst_hbm, out_hbm, src_all, dst_all,
                 rows_a, rows_b, acc_sh, isem0, isem1, sem_a, sem_b,
                 ssem_a, ssem_b):
    c = lax.axis_index("c")
    s = lax.axis_index("s")
    wid = c * NS + s
    rows = [rows_a, rows_b]
    sems = [sem_a, sem_b]
    ssems = [ssem_a, ssem_b]

    ih0 = pltpu.async_copy(src_hbm.at[wid], src_all, isem0)
    ih1 = pltpu.async_copy(dst_hbm.at[wid], dst_all, isem1)

    pltpu.sync_copy(g_hbm.at[pl.ds(s * RPT, RPT)],
                    acc_sh.at[pl.ds(s * RPT, RPT)])

    @pl.when(s == 0)
    def _():
        pltpu.sync_copy(g_hbm.at[pl.ds(TAIL_OFF, TAIL)],
                        acc_sh.at[pl.ds(TAIL_OFF, TAIL)])

    ih0.wait()
    ih1.wait()
    plsc.subcore_barrier()

    def quad(k, carry):
        i0 = k * NBUF
        handles = []
        for b in range(NBUF):
            off = pl.multiple_of((i0 + b) * CH, 8)
            handles.append(
                pltpu.async_copy(g_hbm.at[src_all.at[pl.ds(off, CH)]],
                                 rows[b], sems[b]))
        scat = []
        for b in range(NBUF):
            handles[b].wait()
            scat.append(
                pltpu.async_copy(rows[b], acc_sh.at[dst_all.at[i0 + b]],
                                 ssems[b], add=True))
        for b in range(NBUF):
            scat[b].wait()
        return carry

    lax.fori_loop(0, NQUAD, quad, 0)
    for i in range(NQUAD * NBUF, NCHUNK):
        pltpu.async_copy(g_hbm.at[src_all.at[pl.ds(i * CH, CH)]], rows_a,
                         sem_a).wait()
        pltpu.sync_copy(rows_a, acc_sh.at[dst_all.at[i]], add=True)

    plsc.subcore_barrier()
    pltpu.sync_copy(acc_sh.at[pl.ds(s * RPT, RPT)],
                    out_hbm.at[c, pl.ds(s * RPT, RPT)])

    @pl.when(s == 0)
    def _():
        pltpu.sync_copy(acc_sh.at[pl.ds(TAIL_OFF, TAIL)],
                        out_hbm.at[c, pl.ds(TAIL_OFF, TAIL)])


_sc_agg = functools.partial(
    pl.kernel,
    out_type=jax.ShapeDtypeStruct((NC, N_NODES, HID), jnp.float32),
    mesh=plsc.VectorSubcoreMesh(core_axis_name="c", subcore_axis_name="s"),
    scratch_types=[
        pltpu.VMEM((EPW,), jnp.int32),
        pltpu.VMEM((NCHUNK, CH), jnp.int32),
        pltpu.VMEM((CH, HID), jnp.float32),
        pltpu.VMEM((CH, HID), jnp.float32),
        pltpu.VMEM_SHARED((N_NODES, HID), jnp.float32),
        pltpu.SemaphoreType.DMA,
        pltpu.SemaphoreType.DMA,
        pltpu.SemaphoreType.DMA,
        pltpu.SemaphoreType.DMA,
        pltpu.SemaphoreType.DMA,
        pltpu.SemaphoreType.DMA,
    ],
)(_sc_agg_body)


DW = 16


def _sc_deg_body(ones_hbm, dst_hbm, out_hbm, dst_all, ones_v, acc_sh,
                 isem, sem_a, sem_b, sem_c, sem_d):
    c = lax.axis_index("c")
    s = lax.axis_index("s")
    wid = c * NS + s
    sems = [sem_a, sem_b, sem_c, sem_d]

    ih = pltpu.async_copy(dst_hbm.at[wid], dst_all, isem)
    pltpu.sync_copy(ones_hbm.at[pl.ds(s * RPT, RPT)],
                    acc_sh.at[pl.ds(s * RPT, RPT)])

    @pl.when(s == 0)
    def _():
        pltpu.sync_copy(ones_hbm.at[pl.ds(TAIL_OFF, TAIL)],
                        acc_sh.at[pl.ds(TAIL_OFF, TAIL)])

    pltpu.sync_copy(ones_hbm.at[pl.ds(0, CH)], ones_v)
    ih.wait()
    plsc.subcore_barrier()

    nq = NCHUNK // 4
    def quad(k, carry):
        i0 = k * 4
        handles = [
            pltpu.async_copy(ones_v, acc_sh.at[dst_all.at[i0 + b]], sems[b],
                             add=True)
            for b in range(4)
        ]
        for h in handles:
            h.wait()
        return carry

    lax.fori_loop(0, nq, quad, 0)
    for i in range(nq * 4, NCHUNK):
        pltpu.async_copy(ones_v, acc_sh.at[dst_all.at[i]], sem_a,
                         add=True).wait()

    plsc.subcore_barrier()
    pltpu.sync_copy(acc_sh.at[pl.ds(s * RPT, RPT)],
                    out_hbm.at[c, pl.ds(s * RPT, RPT)])

    @pl.when(s == 0)
    def _():
        pltpu.sync_copy(acc_sh.at[pl.ds(TAIL_OFF, TAIL)],
                        out_hbm.at[c, pl.ds(TAIL_OFF, TAIL)])


_sc_deg = functools.partial(
    pl.kernel,
    out_type=jax.ShapeDtypeStruct((NC, N_NODES, DW), jnp.float32),
    mesh=plsc.VectorSubcoreMesh(core_axis_name="c", subcore_axis_name="s"),
    scratch_types=[
        pltpu.VMEM((NCHUNK, CH), jnp.int32),
        pltpu.VMEM((CH, DW), jnp.float32),
        pltpu.VMEM_SHARED((N_NODES, DW), jnp.float32),
        pltpu.SemaphoreType.DMA,
        pltpu.SemaphoreType.DMA,
        pltpu.SemaphoreType.DMA,
        pltpu.SemaphoreType.DMA,
        pltpu.SemaphoreType.DMA,
    ],
)(_sc_deg_body)


def _tc_embed_body(x_ref, accd_ref, w1_ref, b1_ref, w2_ref, b2_ref, w0_ref,
                   g_ref, dinv_ref):
    deg = accd_ref[0][:, 0:1] + accd_ref[1][:, 0:1] - 1.0
    dinv = 1.0 / jnp.sqrt(deg)
    h = jnp.maximum(
        jnp.dot(x_ref[...], w1_ref[...], preferred_element_type=jnp.float32)
        + b1_ref[...], 0.0)
    h = jnp.dot(h, w2_ref[...], preferred_element_type=jnp.float32) + b2_ref[...]
    g_ref[...] = dinv * jnp.dot(h, w0_ref[...],
                                preferred_element_type=jnp.float32)
    dinv_ref[...] = dinv


def _tc_layer_body(acc_ref, g_ref, dinv_ref, b_ref, w_ref, gout_ref):
    dinv = dinv_ref[...]
    h = acc_ref[0] + acc_ref[1] - g_ref[...]
    h = jnp.maximum(dinv * h + b_ref[...], 0.0)
    gout_ref[...] = dinv * jnp.dot(h, w_ref[...],
                                   preferred_element_type=jnp.float32)


def _tc_final_body(acc_ref, g_ref, dinv_ref, b_ref, bi_ref, wout_ref,
                   bout_ref, out_ref):
    dinv = dinv_ref[...]
    h = acc_ref[0] + acc_ref[1] - g_ref[...]
    h = jnp.maximum(dinv * h + b_ref[...], 0.0)
    onehot = (bi_ref[...] == lax.broadcasted_iota(
        jnp.int32, (N_NODES, NGRAPH), 1)).astype(jnp.float32)
    pooled = lax.dot_general(onehot, h, (((0,), (0,)), ((), ())),
                             preferred_element_type=jnp.float32)
    counts = lax.dot_general(onehot, jnp.ones((N_NODES, 1), jnp.float32),
                             (((0,), (0,)), ((), ())),
                             preferred_element_type=jnp.float32)
    pooled = pooled / jnp.maximum(counts, 1.0)
    out_ref[...] = jnp.dot(pooled, wout_ref[...],
                           preferred_element_type=jnp.float32) + bout_ref[...]


def kernel(x, edge_index, edge_attr, batch_idx, W_emb1, b_emb1, W_emb2,
           b_emb2, W_gcn0, b_gcn0, W_gcn1, b_gcn1, W_gcn2, b_gcn2, W_out,
           b_out):
    del edge_attr
    src = edge_index[0].reshape(NW, EPW)
    dst = edge_index[1].reshape(NW, NCHUNK, CH)
    b1 = b_emb1.reshape(1, HID)
    b2 = b_emb2.reshape(1, HID)
    bg0 = b_gcn0.reshape(1, HID)
    bg1 = b_gcn1.reshape(1, HID)
    bg2 = b_gcn2.reshape(1, HID)
    bo = b_out.reshape(1, 1)
    bi = batch_idx.reshape(N_NODES, 1)

    ones_tab = jnp.ones((N_NODES, DW), jnp.float32)
    acc_deg = _sc_deg(ones_tab, dst)

    g0, dinv = pl.pallas_call(
        _tc_embed_body,
        out_shape=(jax.ShapeDtypeStruct((N_NODES, HID), jnp.float32),
                   jax.ShapeDtypeStruct((N_NODES, 1), jnp.float32)),
    )(x, acc_deg, W_emb1, b1, W_emb2, b2, W_gcn0)

    acc0 = _sc_agg(g0, src, dst)
    g1 = pl.pallas_call(
        _tc_layer_body,
        out_shape=jax.ShapeDtypeStruct((N_NODES, HID), jnp.float32),
    )(acc0, g0, dinv, bg0, W_gcn1)

    acc1 = _sc_agg(g1, src, dst)
    g2 = pl.pallas_call(
        _tc_layer_body,
        out_shape=jax.ShapeDtypeStruct((N_NODES, HID), jnp.float32),
    )(acc1, g1, dinv, bg1, W_gcn2)

    acc2 = _sc_agg(g2, src, dst)
    out = pl.pallas_call(
        _tc_final_body,
        out_shape=jax.ShapeDtypeStruct((NGRAPH, 1), jnp.float32),
    )(acc2, g2, dinv, bg2, bi, W_out, bo)
    return out


# speedup vs baseline: 24.2259x; 1.2265x over previous
import functools

import jax
import jax.numpy as jnp
from jax import lax
from jax.experimental import pallas as pl
from jax.experimental.pallas import tpu as pltpu
from jax.experimental.pallas import tpu_sc as plsc

N_NODES = 10000
N_EDGES = 320000
HID = 128
NGRAPH = 64

_INFO = plsc.get_sparse_core_info()
NC = _INFO.num_cores
NS = _INFO.num_subcores
NW = NC * NS
EPW = N_EDGES // NW
CH = 80
NCHUNK = EPW // CH
NBUF = 3
NRING = NCHUNK // NBUF
RPT = 624
TAIL = N_NODES - RPT * NS
TAIL_OFF = RPT * NS


def _src_at(src_all, i):
    return src_all.at[pl.ds(pl.multiple_of(i * CH, 8), CH)]


def _sc_agg_body(g_hbm, src_hbm, dst_hbm, out_hbm, src_all, dst_all,
                 rows_a, rows_b, rows_c, isem0, isem1,
                 gsem_a, gsem_b, gsem_c, ssem_a, ssem_b, ssem_c, acc_sh):
    c = lax.axis_index("c")
    s = lax.axis_index("s")
    wid = c * NS + s
    rows = [rows_a, rows_b, rows_c]
    gsems = [gsem_a, gsem_b, gsem_c]
    ssems = [ssem_a, ssem_b, ssem_c]

    ih0 = pltpu.async_copy(src_hbm.at[wid], src_all, isem0)
    ih1 = pltpu.async_copy(dst_hbm.at[wid], dst_all, isem1)

    pltpu.sync_copy(g_hbm.at[pl.ds(s * RPT, RPT)],
                    acc_sh.at[pl.ds(s * RPT, RPT)])

    @pl.when(s == 0)
    def _():
        pltpu.sync_copy(g_hbm.at[pl.ds(TAIL_OFF, TAIL)],
                        acc_sh.at[pl.ds(TAIL_OFF, TAIL)])

    ih0.wait()
    ih1.wait()
    plsc.subcore_barrier()

    for b in range(NBUF):
        pltpu.async_copy(g_hbm.at[_src_at(src_all, b)], rows[b], gsems[b])

    def ring(k, carry):
        i0 = k * NBUF
        for b in range(NBUF):
            pltpu.make_async_copy(g_hbm.at[_src_at(src_all, 0)], rows[b],
                                  gsems[b]).wait()
            pltpu.async_copy(
                rows[b], acc_sh.at[dst_all.at[pl.ds(
                    pl.multiple_of((i0 + b) * CH, 8), CH)]],
                ssems[b], add=True)
        for b in range(NBUF):
            pltpu.make_async_copy(
                rows[b], acc_sh.at[dst_all.at[pl.ds(0, CH)]],
                ssems[b]).wait()

            @pl.when(i0 + b + NBUF < NCHUNK)
            def _():
                pltpu.async_copy(
                    g_hbm.at[_src_at(src_all, i0 + b + NBUF)], rows[b],
                    gsems[b])
        return carry

    lax.fori_loop(0, NRING, ring, 0)
    for i in range(NRING * NBUF, NCHUNK):
        b = i - NRING * NBUF
        pltpu.make_async_copy(g_hbm.at[_src_at(src_all, 0)], rows[b],
                              gsems[b]).wait()
        pltpu.sync_copy(rows[b],
                        acc_sh.at[dst_all.at[pl.ds(i * CH, CH)]], add=True)

    plsc.subcore_barrier()
    pltpu.sync_copy(acc_sh.at[pl.ds(s * RPT, RPT)],
                    out_hbm.at[c, pl.ds(s * RPT, RPT)])

    @pl.when(s == 0)
    def _():
        pltpu.sync_copy(acc_sh.at[pl.ds(TAIL_OFF, TAIL)],
                        out_hbm.at[c, pl.ds(TAIL_OFF, TAIL)])


_sc_agg = functools.partial(
    pl.kernel,
    out_type=jax.ShapeDtypeStruct((NC, N_NODES, HID), jnp.float32),
    mesh=plsc.VectorSubcoreMesh(core_axis_name="c", subcore_axis_name="s"),
    scratch_types=[
        pltpu.VMEM((EPW,), jnp.int32),
        pltpu.VMEM((EPW,), jnp.int32),
        pltpu.VMEM((CH, HID), jnp.float32),
        pltpu.VMEM((CH, HID), jnp.float32),
        pltpu.VMEM((CH, HID), jnp.float32),
        pltpu.SemaphoreType.DMA,
        pltpu.SemaphoreType.DMA,
        pltpu.SemaphoreType.DMA,
        pltpu.SemaphoreType.DMA,
        pltpu.SemaphoreType.DMA,
        pltpu.SemaphoreType.DMA,
        pltpu.SemaphoreType.DMA,
        pltpu.SemaphoreType.DMA,
        pltpu.VMEM_SHARED((N_NODES, HID), jnp.float32),
    ],
)(_sc_agg_body)


DW = 16


def _sc_deg_body(ones_hbm, dst_hbm, out_hbm, dst_all, ones_v, acc_sh,
                 isem, sem_a, sem_b, sem_c, sem_d):
    c = lax.axis_index("c")
    s = lax.axis_index("s")
    wid = c * NS + s
    sems = [sem_a, sem_b, sem_c, sem_d]

    ih = pltpu.async_copy(dst_hbm.at[wid], dst_all, isem)
    pltpu.sync_copy(ones_hbm.at[pl.ds(s * RPT, RPT)],
                    acc_sh.at[pl.ds(s * RPT, RPT)])

    @pl.when(s == 0)
    def _():
        pltpu.sync_copy(ones_hbm.at[pl.ds(TAIL_OFF, TAIL)],
                        acc_sh.at[pl.ds(TAIL_OFF, TAIL)])

    pltpu.sync_copy(ones_hbm.at[pl.ds(0, CH)], ones_v)
    ih.wait()
    plsc.subcore_barrier()

    nq = NCHUNK // 4
    def quad(k, carry):
        i0 = k * 4
        handles = [
            pltpu.async_copy(ones_v, acc_sh.at[dst_all.at[i0 + b]], sems[b],
                             add=True)
            for b in range(4)
        ]
        for h in handles:
            h.wait()
        return carry

    lax.fori_loop(0, nq, quad, 0)
    for i in range(nq * 4, NCHUNK):
        pltpu.async_copy(ones_v, acc_sh.at[dst_all.at[i]], sem_a,
                         add=True).wait()

    plsc.subcore_barrier()
    pltpu.sync_copy(acc_sh.at[pl.ds(s * RPT, RPT)],
                    out_hbm.at[c, pl.ds(s * RPT, RPT)])

    @pl.when(s == 0)
    def _():
        pltpu.sync_copy(acc_sh.at[pl.ds(TAIL_OFF, TAIL)],
                        out_hbm.at[c, pl.ds(TAIL_OFF, TAIL)])


_sc_deg = functools.partial(
    pl.kernel,
    out_type=jax.ShapeDtypeStruct((NC, N_NODES, DW), jnp.float32),
    mesh=plsc.VectorSubcoreMesh(core_axis_name="c", subcore_axis_name="s"),
    scratch_types=[
        pltpu.VMEM((NCHUNK, CH), jnp.int32),
        pltpu.VMEM((CH, DW), jnp.float32),
        pltpu.VMEM_SHARED((N_NODES, DW), jnp.float32),
        pltpu.SemaphoreType.DMA,
        pltpu.SemaphoreType.DMA,
        pltpu.SemaphoreType.DMA,
        pltpu.SemaphoreType.DMA,
        pltpu.SemaphoreType.DMA,
    ],
)(_sc_deg_body)


def _tc_embed_body(x_ref, accd_ref, w1_ref, b1_ref, w2_ref, b2_ref, w0_ref,
                   g_ref, dinv_ref):
    deg = accd_ref[0][:, 0:1] + accd_ref[1][:, 0:1] - 1.0
    dinv = 1.0 / jnp.sqrt(deg)
    h = jnp.maximum(
        jnp.dot(x_ref[...], w1_ref[...], preferred_element_type=jnp.float32)
        + b1_ref[...], 0.0)
    h = jnp.dot(h, w2_ref[...], preferred_element_type=jnp.float32) + b2_ref[...]
    g_ref[...] = dinv * jnp.dot(h, w0_ref[...],
                                preferred_element_type=jnp.float32)
    dinv_ref[...] = dinv


def _tc_layer_body(acc_ref, g_ref, dinv_ref, b_ref, w_ref, gout_ref):
    dinv = dinv_ref[...]
    h = acc_ref[0] + acc_ref[1] - g_ref[...]
    h = jnp.maximum(dinv * h + b_ref[...], 0.0)
    gout_ref[...] = dinv * jnp.dot(h, w_ref[...],
                                   preferred_element_type=jnp.float32)


def _tc_final_body(acc_ref, g_ref, dinv_ref, b_ref, bi_ref, wout_ref,
                   bout_ref, out_ref):
    dinv = dinv_ref[...]
    h = acc_ref[0] + acc_ref[1] - g_ref[...]
    h = jnp.maximum(dinv * h + b_ref[...], 0.0)
    onehot = (bi_ref[...] == lax.broadcasted_iota(
        jnp.int32, (N_NODES, NGRAPH), 1)).astype(jnp.float32)
    pooled = lax.dot_general(onehot, h, (((0,), (0,)), ((), ())),
                             preferred_element_type=jnp.float32)
    counts = lax.dot_general(onehot, jnp.ones((N_NODES, 1), jnp.float32),
                             (((0,), (0,)), ((), ())),
                             preferred_element_type=jnp.float32)
    pooled = pooled / jnp.maximum(counts, 1.0)
    out_ref[...] = jnp.dot(pooled, wout_ref[...],
                           preferred_element_type=jnp.float32) + bout_ref[...]


def kernel(x, edge_index, edge_attr, batch_idx, W_emb1, b_emb1, W_emb2,
           b_emb2, W_gcn0, b_gcn0, W_gcn1, b_gcn1, W_gcn2, b_gcn2, W_out,
           b_out):
    del edge_attr
    src = edge_index[0].reshape(NW, EPW)
    dst = edge_index[1].reshape(NW, EPW)
    dst2d = edge_index[1].reshape(NW, NCHUNK, CH)
    b1 = b_emb1.reshape(1, HID)
    b2 = b_emb2.reshape(1, HID)
    bg0 = b_gcn0.reshape(1, HID)
    bg1 = b_gcn1.reshape(1, HID)
    bg2 = b_gcn2.reshape(1, HID)
    bo = b_out.reshape(1, 1)
    bi = batch_idx.reshape(N_NODES, 1)

    ones_tab = jnp.ones((N_NODES, DW), jnp.float32)
    acc_deg = _sc_deg(ones_tab, dst2d)

    g0, dinv = pl.pallas_call(
        _tc_embed_body,
        out_shape=(jax.ShapeDtypeStruct((N_NODES, HID), jnp.float32),
                   jax.ShapeDtypeStruct((N_NODES, 1), jnp.float32)),
    )(x, acc_deg, W_emb1, b1, W_emb2, b2, W_gcn0)

    acc0 = _sc_agg(g0, src, dst)
    g1 = pl.pallas_call(
        _tc_layer_body,
        out_shape=jax.ShapeDtypeStruct((N_NODES, HID), jnp.float32),
    )(acc0, g0, dinv, bg0, W_gcn1)

    acc1 = _sc_agg(g1, src, dst)
    g2 = pl.pallas_call(
        _tc_layer_body,
        out_shape=jax.ShapeDtypeStruct((N_NODES, HID), jnp.float32),
    )(acc1, g1, dinv, bg1, W_gcn2)

    acc2 = _sc_agg(g2, src, dst)
    out = pl.pallas_call(
        _tc_final_body,
        out_shape=jax.ShapeDtypeStruct((NGRAPH, 1), jnp.float32),
    )(acc2, g2, dinv, bg2, bi, W_out, bo)
    return out


# speedup vs baseline: 24.3005x; 1.0031x over previous
import functools

import jax
import jax.numpy as jnp
from jax import lax
from jax.experimental import pallas as pl
from jax.experimental.pallas import tpu as pltpu
from jax.experimental.pallas import tpu_sc as plsc

N_NODES = 10000
N_EDGES = 320000
HID = 128
NGRAPH = 64

_INFO = plsc.get_sparse_core_info()
NC = _INFO.num_cores
NS = _INFO.num_subcores
NW = NC * NS
EPW = N_EDGES // NW
CH = 80
NCHUNK = EPW // CH
NBUF = 3
NRING = NCHUNK // NBUF
RPT = 624
TAIL = N_NODES - RPT * NS
TAIL_OFF = RPT * NS


def _src_at(src_all, i):
    return src_all.at[pl.ds(pl.multiple_of(i * CH, 8), CH)]


def _sc_agg_body(g_hbm, src_hbm, dst_hbm, out_hbm, src_all, dst_all,
                 rows_a, rows_b, rows_c, isem0, isem1,
                 gsem_a, gsem_b, gsem_c, ssem_a, ssem_b, ssem_c, acc_sh):
    c = lax.axis_index("c")
    s = lax.axis_index("s")
    wid = c * NS + s
    rows = [rows_a, rows_b, rows_c]
    gsems = [gsem_a, gsem_b, gsem_c]
    ssems = [ssem_a, ssem_b, ssem_c]

    ih0 = pltpu.async_copy(src_hbm.at[wid], src_all, isem0)
    ih1 = pltpu.async_copy(dst_hbm.at[wid], dst_all, isem1)

    pltpu.sync_copy(g_hbm.at[pl.ds(s * RPT, RPT)],
                    acc_sh.at[pl.ds(s * RPT, RPT)])

    @pl.when(s == 0)
    def _():
        pltpu.sync_copy(g_hbm.at[pl.ds(TAIL_OFF, TAIL)],
                        acc_sh.at[pl.ds(TAIL_OFF, TAIL)])

    ih0.wait()
    ih1.wait()
    plsc.subcore_barrier()

    for b in range(NBUF):
        pltpu.async_copy(g_hbm.at[_src_at(src_all, b)], rows[b], gsems[b])

    def ring(k, carry):
        i0 = k * NBUF
        for b in range(NBUF):
            pltpu.make_async_copy(g_hbm.at[_src_at(src_all, 0)], rows[b],
                                  gsems[b]).wait()
            pltpu.async_copy(
                rows[b], acc_sh.at[dst_all.at[pl.ds(
                    pl.multiple_of((i0 + b) * CH, 8), CH)]],
                ssems[b], add=True)
        for b in range(NBUF):
            pltpu.make_async_copy(
                rows[b], acc_sh.at[dst_all.at[pl.ds(0, CH)]],
                ssems[b]).wait()

            @pl.when(i0 + b + NBUF < NCHUNK)
            def _():
                pltpu.async_copy(
                    g_hbm.at[_src_at(src_all, i0 + b + NBUF)], rows[b],
                    gsems[b])
        return carry

    lax.fori_loop(0, NRING, ring, 0)
    for i in range(NRING * NBUF, NCHUNK):
        b = i - NRING * NBUF
        pltpu.make_async_copy(g_hbm.at[_src_at(src_all, 0)], rows[b],
                              gsems[b]).wait()
        pltpu.sync_copy(rows[b],
                        acc_sh.at[dst_all.at[pl.ds(i * CH, CH)]], add=True)

    plsc.subcore_barrier()
    pltpu.sync_copy(acc_sh.at[pl.ds(s * RPT, RPT)],
                    out_hbm.at[c, pl.ds(s * RPT, RPT)])

    @pl.when(s == 0)
    def _():
        pltpu.sync_copy(acc_sh.at[pl.ds(TAIL_OFF, TAIL)],
                        out_hbm.at[c, pl.ds(TAIL_OFF, TAIL)])


_sc_agg = functools.partial(
    pl.kernel,
    out_type=jax.ShapeDtypeStruct((NC, N_NODES, HID), jnp.float32),
    mesh=plsc.VectorSubcoreMesh(core_axis_name="c", subcore_axis_name="s"),
    scratch_types=[
        pltpu.VMEM((EPW,), jnp.int32),
        pltpu.VMEM((EPW,), jnp.int32),
        pltpu.VMEM((CH, HID), jnp.float32),
        pltpu.VMEM((CH, HID), jnp.float32),
        pltpu.VMEM((CH, HID), jnp.float32),
        pltpu.SemaphoreType.DMA,
        pltpu.SemaphoreType.DMA,
        pltpu.SemaphoreType.DMA,
        pltpu.SemaphoreType.DMA,
        pltpu.SemaphoreType.DMA,
        pltpu.SemaphoreType.DMA,
        pltpu.SemaphoreType.DMA,
        pltpu.SemaphoreType.DMA,
        pltpu.VMEM_SHARED((N_NODES, HID), jnp.float32),
    ],
)(_sc_agg_body)


DW = 16


def _sc_deg_body(ones_hbm, dst_hbm, out_hbm, dst_all, ones_v, acc_sh,
                 isem, sem_a, sem_b, sem_c, sem_d):
    c = lax.axis_index("c")
    s = lax.axis_index("s")
    wid = c * NS + s
    sems = [sem_a, sem_b, sem_c, sem_d]

    ih = pltpu.async_copy(dst_hbm.at[wid], dst_all, isem)
    pltpu.sync_copy(ones_hbm.at[pl.ds(s * RPT, RPT)],
                    acc_sh.at[pl.ds(s * RPT, RPT)])

    @pl.when(s == 0)
    def _():
        pltpu.sync_copy(ones_hbm.at[pl.ds(TAIL_OFF, TAIL)],
                        acc_sh.at[pl.ds(TAIL_OFF, TAIL)])

    pltpu.sync_copy(ones_hbm.at[pl.ds(0, CH)], ones_v)
    ih.wait()
    plsc.subcore_barrier()

    nq = NCHUNK // 4
    def quad(k, carry):
        i0 = k * 4
        handles = [
            pltpu.async_copy(ones_v, acc_sh.at[dst_all.at[i0 + b]], sems[b],
                             add=True)
            for b in range(4)
        ]
        for h in handles:
            h.wait()
        return carry

    lax.fori_loop(0, nq, quad, 0)
    for i in range(nq * 4, NCHUNK):
        pltpu.async_copy(ones_v, acc_sh.at[dst_all.at[i]], sem_a,
                         add=True).wait()

    plsc.subcore_barrier()
    pltpu.sync_copy(acc_sh.at[pl.ds(s * RPT, RPT)],
                    out_hbm.at[c, pl.ds(s * RPT, RPT)])

    @pl.when(s == 0)
    def _():
        pltpu.sync_copy(acc_sh.at[pl.ds(TAIL_OFF, TAIL)],
                        out_hbm.at[c, pl.ds(TAIL_OFF, TAIL)])


_sc_deg = functools.partial(
    pl.kernel,
    out_type=jax.ShapeDtypeStruct((NC, N_NODES, DW), jnp.float32),
    mesh=plsc.VectorSubcoreMesh(core_axis_name="c", subcore_axis_name="s"),
    scratch_types=[
        pltpu.VMEM((NCHUNK, CH), jnp.int32),
        pltpu.VMEM((CH, DW), jnp.float32),
        pltpu.VMEM_SHARED((N_NODES, DW), jnp.float32),
        pltpu.SemaphoreType.DMA,
        pltpu.SemaphoreType.DMA,
        pltpu.SemaphoreType.DMA,
        pltpu.SemaphoreType.DMA,
        pltpu.SemaphoreType.DMA,
    ],
)(_sc_deg_body)


def _tc_mlp_body(x_ref, w1_ref, b1_ref, w2_ref, b2_ref, h_ref):
    h = jnp.maximum(
        jnp.dot(x_ref[...], w1_ref[...], preferred_element_type=jnp.float32)
        + b1_ref[...], 0.0)
    h_ref[...] = (jnp.dot(h, w2_ref[...], preferred_element_type=jnp.float32)
                  + b2_ref[...])


def _tc_g0_body(h_ref, accd_ref, w0_ref, g_ref, dinv_ref):
    deg = accd_ref[0][:, 0:1] + accd_ref[1][:, 0:1] - 1.0
    dinv = 1.0 / jnp.sqrt(deg)
    g_ref[...] = dinv * jnp.dot(h_ref[...], w0_ref[...],
                                preferred_element_type=jnp.float32)
    dinv_ref[...] = dinv


def _tc_layer_body(acc_ref, g_ref, dinv_ref, b_ref, w_ref, gout_ref):
    dinv = dinv_ref[...]
    h = acc_ref[0] + acc_ref[1] - g_ref[...]
    h = jnp.maximum(dinv * h + b_ref[...], 0.0)
    gout_ref[...] = dinv * jnp.dot(h, w_ref[...],
                                   preferred_element_type=jnp.float32)


def _tc_final_body(acc_ref, g_ref, dinv_ref, b_ref, bi_ref, wout_ref,
                   bout_ref, out_ref):
    dinv = dinv_ref[...]
    h = acc_ref[0] + acc_ref[1] - g_ref[...]
    h = jnp.maximum(dinv * h + b_ref[...], 0.0)
    onehot = (bi_ref[...] == lax.broadcasted_iota(
        jnp.int32, (N_NODES, NGRAPH), 1)).astype(jnp.float32)
    pooled = lax.dot_general(onehot, h, (((0,), (0,)), ((), ())),
                             preferred_element_type=jnp.float32)
    counts = lax.dot_general(onehot, jnp.ones((N_NODES, 1), jnp.float32),
                             (((0,), (0,)), ((), ())),
                             preferred_element_type=jnp.float32)
    pooled = pooled / jnp.maximum(counts, 1.0)
    out_ref[...] = jnp.dot(pooled, wout_ref[...],
                           preferred_element_type=jnp.float32) + bout_ref[...]


def kernel(x, edge_index, edge_attr, batch_idx, W_emb1, b_emb1, W_emb2,
           b_emb2, W_gcn0, b_gcn0, W_gcn1, b_gcn1, W_gcn2, b_gcn2, W_out,
           b_out):
    del edge_attr
    src = edge_index[0].reshape(NW, EPW)
    dst = edge_index[1].reshape(NW, EPW)
    dst2d = edge_index[1].reshape(NW, NCHUNK, CH)
    b1 = b_emb1.reshape(1, HID)
    b2 = b_emb2.reshape(1, HID)
    bg0 = b_gcn0.reshape(1, HID)
    bg1 = b_gcn1.reshape(1, HID)
    bg2 = b_gcn2.reshape(1, HID)
    bo = b_out.reshape(1, 1)
    bi = batch_idx.reshape(N_NODES, 1)

    ones_tab = jnp.ones((N_NODES, DW), jnp.float32)
    acc_deg = _sc_deg(ones_tab, dst2d)

    h0 = pl.pallas_call(
        _tc_mlp_body,
        out_shape=jax.ShapeDtypeStruct((N_NODES, HID), jnp.float32),
    )(x, W_emb1, b1, W_emb2, b2)

    g0, dinv = pl.pallas_call(
        _tc_g0_body,
        out_shape=(jax.ShapeDtypeStruct((N_NODES, HID), jnp.float32),
                   jax.ShapeDtypeStruct((N_NODES, 1), jnp.float32)),
    )(h0, acc_deg, W_gcn0)

    acc0 = _sc_agg(g0, src, dst)
    g1 = pl.pallas_call(
        _tc_layer_body,
        out_shape=jax.ShapeDtypeStruct((N_NODES, HID), jnp.float32),
    )(acc0, g0, dinv, bg0, W_gcn1)

    acc1 = _sc_agg(g1, src, dst)
    g2 = pl.pallas_call(
        _tc_layer_body,
        out_shape=jax.ShapeDtypeStruct((N_NODES, HID), jnp.float32),
    )(acc1, g1, dinv, bg1, W_gcn2)

    acc2 = _sc_agg(g2, src, dst)
    out = pl.pallas_call(
        _tc_final_body,
        out_shape=jax.ShapeDtypeStruct((NGRAPH, 1), jnp.float32),
    )(acc2, g2, dinv, bg2, bi, W_out, bo)
    return out
